# baseline (speedup 1.0000x reference)
import functools

import jax
import jax.numpy as jnp
from jax import lax
from jax.experimental import pallas as pl
from jax.experimental.pallas import tpu as pltpu
from jax.experimental.pallas import tpu_sc as plsc

_INTERPRET = False

NUM_GATES_K = 64
EPS_K = 1e-09
BN = 512
CHUNK = 16


def _gate_kernel(nb, cap, n_tok, nbatch, x_ref, w_ref, t1_ref, v1_ref,
                 t2_ref, v2_ref, loss_ref, i2_s, p2_s, g2_s, cnt_s, prx_s,
                 carry1, carry2):
    i = pl.program_id(0)
    E = NUM_GATES_K
    capf = float(cap)
    bi = i // nb
    ni = lax.rem(i, nb)

    @pl.when(i == 0)
    def _():
        cnt_s[...] = jnp.zeros_like(cnt_s)
        prx_s[...] = jnp.zeros_like(prx_s)

    @pl.when(ni == 0)
    def _():
        carry1[...] = jnp.zeros_like(carry1)
        carry2[...] = jnp.zeros_like(carry2)

    xb = x_ref[0]
    w = w_ref[...]
    logits = jnp.dot(xb, w, preferred_element_type=jnp.float32)
    probs = jax.nn.softmax(logits, axis=-1)

    e_iota = lax.broadcasted_iota(jnp.int32, (BN, E), 1)
    g1 = jnp.max(probs, axis=-1, keepdims=True)
    i1 = jnp.min(jnp.where(probs == g1, e_iota, E), axis=-1, keepdims=True)
    m1 = (e_iota == i1).astype(jnp.float32)
    pnt = probs * (1.0 - m1)
    g2 = jnp.max(pnt, axis=-1, keepdims=True)
    i2 = jnp.min(jnp.where(pnt == g2, e_iota, E), axis=-1, keepdims=True)
    m2 = (e_iota == i2).astype(jnp.float32)

    denom = g1 + g2 + EPS_K
    g1n = g1 / denom
    g2n = g2 / denom

    r_io = lax.broadcasted_iota(jnp.int32, (BN, BN), 0)
    c_io = lax.broadcasted_iota(jnp.int32, (BN, BN), 1)
    tri = (c_io < r_io).astype(jnp.float32)

    ex1 = jnp.dot(tri, m1, preferred_element_type=jnp.float32) + carry1[...]
    pos1 = jnp.sum(ex1 * m1, axis=-1, keepdims=True)
    kept1 = pos1 < capf
    v1 = jnp.where(kept1, g1n, 0.0)
    t1 = jnp.where(kept1, i1 * cap + pos1.astype(jnp.int32), -1)

    ex2 = jnp.dot(tri, m2, preferred_element_type=jnp.float32) + carry2[...]
    p2 = jnp.sum(ex2 * m2, axis=-1, keepdims=True)

    carry1[...] = carry1[...] + jnp.sum(m1, axis=0, keepdims=True)
    carry2[...] = carry2[...] + jnp.sum(m2, axis=0, keepdims=True)
    cnt_s[bi] = cnt_s[bi] + jnp.sum(m1, axis=0, keepdims=True)
    prx_s[bi] = prx_s[bi] + jnp.sum(probs, axis=0, keepdims=True)

    t1_ref[...] = t1.reshape(1, 1, BN)
    v1_ref[...] = v1.reshape(1, 1, BN)
    i2_s[i] = i2.reshape(1, BN)
    p2_s[i] = p2.reshape(1, BN)
    g2_s[i] = g2n.reshape(1, BN)

    @pl.when(ni == nb - 1)
    def _finalize():
        cnt_b = jnp.minimum(cnt_s[bi], capf)
        for k in range(nb):
            i2k = i2_s[bi * nb + k].reshape(BN, 1)
            p2k = p2_s[bi * nb + k].reshape(BN, 1)
            g2k = g2_s[bi * nb + k].reshape(BN, 1)
            m2k = e_iota == i2k
            m1cnt = jnp.sum(jnp.where(m2k, cnt_b, 0.0), axis=-1,
                            keepdims=True)
            pos2 = p2k + m1cnt
            kept2 = pos2 < capf
            v2 = jnp.where(kept2, g2k, 0.0)
            t2 = jnp.where(kept2, i2k * cap + pos2.astype(jnp.int32), -1)
            t2_ref[0, k] = t2.reshape(1, BN)
            v2_ref[0, k] = v2.reshape(1, BN)

    @pl.when(i == pl.num_programs(0) - 1)
    def _loss():
        s = jnp.sum(cnt_s[...] * prx_s[...])
        scale = float(E * E) / (float(n_tok) * float(n_tok) * nbatch * E)
        loss_ref[...] = (s * scale).reshape(1, 1)


def _emit_sc_kernel(rows_per_w, cap, nc, t1_hbm, v1_hbm, t2_hbm, v2_hbm,
                    disp_hbm, comb_hbm, t1_ts, v1_ts, t2_ts, v2_ts,
                    disp_buf, comb_buf):
    E = NUM_GATES_K
    row_w = E * cap
    wid = lax.axis_index("s") * nc + lax.axis_index("c")
    base_row = wid * rows_per_w

    pltpu.sync_copy(t1_hbm.at[pl.ds(base_row, rows_per_w)], t1_ts)
    pltpu.sync_copy(v1_hbm.at[pl.ds(base_row, rows_per_w)], v1_ts)
    pltpu.sync_copy(t2_hbm.at[pl.ds(base_row, rows_per_w)], t2_ts)
    pltpu.sync_copy(v2_hbm.at[pl.ds(base_row, rows_per_w)], v2_ts)

    zeros = jnp.zeros((16,), jnp.float32)

    def _zero_body(k, carry):
        disp_buf[pl.ds(k * 16, 16)] = zeros
        comb_buf[pl.ds(k * 16, 16)] = zeros
        return carry

    lax.fori_loop(0, (CHUNK * row_w) // 16, _zero_body, 0)

    r_iota = lax.iota(jnp.int32, 16)
    ones = jnp.ones((16,), jnp.float32)

    for c in range(rows_per_w // CHUNK):
        off = c * CHUNK
        t1c = t1_ts[pl.ds(off, CHUNK)]
        v1c = v1_ts[pl.ds(off, CHUNK)]
        t2c = t2_ts[pl.ds(off, CHUNK)]
        v2c = v2_ts[pl.ds(off, CHUNK)]

        mask1 = t1c >= 0
        lidx1 = r_iota * row_w + jnp.maximum(t1c, 0)
        mask2 = t2c >= 0
        lidx2 = r_iota * row_w + jnp.maximum(t2c, 0)
        maskd2 = v2c > 0.0

        plsc.store_scatter(comb_buf, [lidx1], v1c, mask=mask1)
        plsc.store_scatter(comb_buf, [lidx2], v2c, mask=mask2)
        plsc.store_scatter(disp_buf, [lidx1], ones, mask=mask1)
        plsc.store_scatter(disp_buf, [lidx2], ones, mask=maskd2)

        gbase = (base_row + off) * row_w
        pltpu.sync_copy(disp_buf, disp_hbm.at[pl.ds(gbase, CHUNK * row_w)])
        pltpu.sync_copy(comb_buf, comb_hbm.at[pl.ds(gbase, CHUNK * row_w)])

        plsc.store_scatter(comb_buf, [lidx1], zeros, mask=mask1)
        plsc.store_scatter(comb_buf, [lidx2], zeros, mask=mask2)
        plsc.store_scatter(disp_buf, [lidx1], zeros, mask=mask1)
        plsc.store_scatter(disp_buf, [lidx2], zeros, mask=maskd2)


@jax.jit
def kernel(x, w_gating):
    b, n, d = x.shape
    E = NUM_GATES_K
    cap = max(min(n, int(n * 1.25 / E)), 4)
    nb = n // BN
    grid0 = b * nb

    tok_spec = pl.BlockSpec((1, 1, BN), lambda i: (i, 0, 0))
    row_spec = pl.BlockSpec((1, nb, 1, BN), lambda i, nb=nb: (i // nb, 0, 0, 0))

    outs = pl.pallas_call(
        functools.partial(_gate_kernel, nb, cap, n, b),
        grid=(grid0,),
        in_specs=[
            pl.BlockSpec((1, BN, d), lambda i, nb=nb: (i // nb, i % nb, 0)),
            pl.BlockSpec((d, E), lambda i: (0, 0)),
        ],
        out_specs=[tok_spec, tok_spec, row_spec, row_spec,
                   pl.BlockSpec((1, 1), lambda i: (0, 0))],
        out_shape=[
            jax.ShapeDtypeStruct((grid0, 1, BN), jnp.int32),
            jax.ShapeDtypeStruct((grid0, 1, BN), jnp.float32),
            jax.ShapeDtypeStruct((b, nb, 1, BN), jnp.int32),
            jax.ShapeDtypeStruct((b, nb, 1, BN), jnp.float32),
            jax.ShapeDtypeStruct((1, 1), jnp.float32),
        ],
        scratch_shapes=[pltpu.VMEM((grid0, 1, BN), jnp.int32),
                        pltpu.VMEM((grid0, 1, BN), jnp.float32),
                        pltpu.VMEM((grid0, 1, BN), jnp.float32),
                        pltpu.VMEM((b, 1, E), jnp.float32),
                        pltpu.VMEM((b, 1, E), jnp.float32),
                        pltpu.VMEM((1, E), jnp.float32),
                        pltpu.VMEM((1, E), jnp.float32)],
        interpret=_INTERPRET,
    )(x, w_gating)
    t1, v1, t2, v2, loss = outs

    tot = b * n
    t1 = t1.reshape(tot)
    v1 = v1.reshape(tot)
    t2 = t2.reshape(tot)
    v2 = v2.reshape(tot)

    info = plsc.get_sparse_core_info()
    nw = info.num_cores * info.num_subcores
    rows_per_w = tot // nw
    row_w = E * cap

    sc_emit = pl.kernel(
        functools.partial(_emit_sc_kernel, rows_per_w, cap, info.num_cores),
        mesh=plsc.VectorSubcoreMesh(core_axis_name="c", subcore_axis_name="s"),
        out_type=[
            jax.ShapeDtypeStruct((tot * row_w,), jnp.float32),
            jax.ShapeDtypeStruct((tot * row_w,), jnp.float32),
        ],
        scratch_types=[
            pltpu.VMEM((rows_per_w,), jnp.int32),
            pltpu.VMEM((rows_per_w,), jnp.float32),
            pltpu.VMEM((rows_per_w,), jnp.int32),
            pltpu.VMEM((rows_per_w,), jnp.float32),
            pltpu.VMEM((CHUNK * row_w,), jnp.float32),
            pltpu.VMEM((CHUNK * row_w,), jnp.float32),
        ],
        compiler_params=pltpu.CompilerParams(needs_layout_passes=False),
    )
    disp, comb = sc_emit(t1, v1, t2, v2)

    dispatch = disp.reshape(b, n, E, cap)
    combine = comb.reshape(b, n, E, cap)
    return dispatch, combine, loss.reshape(())
